# baseline (speedup 1.0000x reference)
import functools

import jax
import jax.numpy as jnp
from jax import lax
from jax.experimental import pallas as pl
from jax.experimental.pallas import tpu as pltpu
from jax.experimental.pallas import tpu_sc as plsc

_S = 4096
_R = 26
_D = 64
_NC, _NS = 2, 16
_NW = _NC * _NS
_SB = _S // _NW
_RC = 13
_CHUNK = _RC * _SB


def _embed_body(xt_hbm, table_hbm, out_hbm, idx_v, rows_v, big_v,
                gsem, wsem_a, wsem_b, isem):
    w = lax.axis_index("s") * _NC + lax.axis_index("c")
    s0 = w * _SB
    lane = jax.lax.iota(jnp.int32, 16)
    dstc = [16 * m + lane for m in range(8)]

    for c in range(2):
        ihs = [
            pltpu.async_copy(
                xt_hbm.at[_RC * c + j, pl.ds(s0, _SB)],
                idx_v.at[pl.ds(_SB * j, _SB)],
                isem,
            )
            for j in range(_RC)
        ]
        for h in ihs:
            h.wait()
        pltpu.async_copy(table_hbm.at[idx_v], rows_v, gsem).wait()

        def tile_loop(j, carry, c=c):
            r = _RC * c + j
            rb = [j * _SB + 16 * m + lane for m in range(8)]
            for half in range(2):
                wsem = wsem_a if half == 0 else wsem_b

                @pl.when(j > 0)
                def _(half=half, wsem=wsem):
                    for ti in range(4 * half, 4 * half + 4):
                        pltpu.make_async_copy(
                            out_hbm.at[0],
                            big_v.at[pl.ds(8 * ti, 8), :],
                            wsem,
                        ).wait()

                for q in (2 * half, 2 * half + 1):
                    def diag_loop(i, cc, q=q, rb=rb):
                        dg = q * 16 + ((i + lane) & 15)
                        for m0 in range(0, 8, 4):
                            vs = [
                                plsc.load_gather(rows_v, [rb[m0 + t], dg])
                                for t in range(4)
                            ]
                            for t in range(4):
                                plsc.store_scatter(
                                    big_v, [dg, dstc[m0 + t]], vs[t]
                                )
                        return cc
                    lax.fori_loop(0, 16, diag_loop, 0)
                for ti in range(4 * half, 4 * half + 4):
                    pltpu.async_copy(
                        big_v.at[pl.ds(8 * ti, 8), :],
                        out_hbm.at[r * 256 + ti * 32 + w],
                        wsem,
                    )
            return carry

        lax.fori_loop(0, _RC, tile_loop, 0)
        for ti in range(8):
            pltpu.make_async_copy(
                out_hbm.at[0],
                big_v.at[pl.ds(8 * ti, 8), :],
                wsem_a if ti < 4 else wsem_b,
            ).wait()


def _pack_body(wt_hbm, tail_hbm, out2_hbm, in_a, in_b, out_a, out_b,
               inx_v, outx_v, tail_v, si_a, si_b, so_a, so_b):
    w = lax.axis_index("s") * _NC + lax.axis_index("c")
    lane = jax.lax.iota(jnp.int32, 16)
    rowc = [16 * m + lane for m in range(4)]
    scol = [[64 * a + 16 * q + lane for q in range(4)] for a in (0, 1)]

    def transpose_pair(i2, cc, in_v=None, out_v=None):
        for ii in range(2):
            i3 = 2 * i2 + ii
            usub = i3 >> 4
            dgs = (i3 + lane) & 15
            dg2 = 2 * dgs
            for p0 in (0, 16, 32, 48):
                rowv2 = dgs + (64 * usub + p0)
                for a in (0, 1):
                    col = dg2 + (128 * usub + 2 * p0 + a)
                    vs = [
                        plsc.load_gather(in_v, [rowc[q], col])
                        for q in range(4)
                    ]
                    for q in range(4):
                        plsc.store_scatter(
                            out_v, [rowv2, scol[a][q]], vs[q]
                        )
        return cc

    h_in, h_out = {}, {}
    bufs = [(in_a, out_a, si_a, so_a), (in_b, out_b, si_b, so_b)]
    c_base = w * 24 * 128
    h_in[0] = pltpu.async_copy(
        wt_hbm.at[:, pl.ds(c_base, 256)], in_a, si_a
    )
    for mm in range(12):
        in_v, out_v, si, so = bufs[mm % 2]
        if mm < 11:
            n_in = bufs[(mm + 1) % 2][0]
            h_in[mm + 1] = pltpu.async_copy(
                wt_hbm.at[:, pl.ds(c_base + (mm + 1) * 256, 256)],
                n_in,
                bufs[(mm + 1) % 2][2],
            )
        h_in[mm].wait()
        if mm >= 2:
            h_out[mm - 2].wait()
        lax.fori_loop(
            0, 16,
            functools.partial(transpose_pair, in_v=in_v, out_v=out_v),
            0,
        )
        h_out[mm] = pltpu.async_copy(
            out_v,
            out2_hbm.at[pl.ds((w * 24 + 2 * mm) * 64, 128), :],
            so,
        )
    h_out[10].wait()
    h_out[11].wait()

    @pl.when(w < 13)
    def _():
        pltpu.async_copy(
            wt_hbm.at[:, pl.ds((768 + w) * 128, 128)], inx_v, si_a
        ).wait()
        lax.fori_loop(
            0, 8,
            functools.partial(transpose_pair, in_v=inx_v, out_v=outx_v),
            0,
        )
        pltpu.async_copy(
            outx_v, out2_hbm.at[pl.ds((768 + w) * 64, 64), :], so_a
        ).wait()

    @pl.when(w == 31)
    def _():
        pltpu.async_copy(tail_hbm, tail_v, si_b).wait()
        pltpu.async_copy(
            tail_v, out2_hbm.at[pl.ds(49984, 16), :], so_b
        ).wait()


def _pack_table(weight):
    wt = weight.T
    tail = lax.slice(weight, (99968, 0), (100000, 64)).reshape(16, 128)
    mesh = plsc.VectorSubcoreMesh(core_axis_name="c", subcore_axis_name="s")
    k = pl.kernel(
        _pack_body,
        mesh=mesh,
        out_type=jax.ShapeDtypeStruct((50000, 128), jnp.float32),
        scratch_types=[
            pltpu.VMEM((_D, 256), jnp.float32),
            pltpu.VMEM((_D, 256), jnp.float32),
            pltpu.VMEM((128, 128), jnp.float32),
            pltpu.VMEM((128, 128), jnp.float32),
            pltpu.VMEM((_D, 128), jnp.float32),
            pltpu.VMEM((_D, 128), jnp.float32),
            pltpu.VMEM((16, 128), jnp.float32),
            pltpu.SemaphoreType.DMA,
            pltpu.SemaphoreType.DMA,
            pltpu.SemaphoreType.DMA,
            pltpu.SemaphoreType.DMA,
        ],
        compiler_params=pltpu.CompilerParams(
            use_tc_tiling_on_sc=True, needs_layout_passes=False
        ),
    )
    return k(wt, tail).reshape(100000, _D)


def kernel(x, weight):
    xt = x.T
    weight = _pack_table(weight)
    mesh = plsc.VectorSubcoreMesh(core_axis_name="c", subcore_axis_name="s")
    k = pl.kernel(
        _embed_body,
        mesh=mesh,
        out_type=jax.ShapeDtypeStruct((_R * 8 * _NW, 8, 128), jnp.float32),
        scratch_types=[
            pltpu.VMEM((_CHUNK,), jnp.int32),
            pltpu.VMEM((_CHUNK, _D), jnp.float32),
            pltpu.VMEM((_D, 128), jnp.float32),
            pltpu.SemaphoreType.DMA,
            pltpu.SemaphoreType.DMA,
            pltpu.SemaphoreType.DMA,
            pltpu.SemaphoreType.DMA,
        ],
        compiler_params=pltpu.CompilerParams(
            use_tc_tiling_on_sc=False, needs_layout_passes=False
        ),
    )
    out3 = k(xt, weight)
    t = out3.reshape(_R, 8, _NW, 8, 128)
    return t.transpose(2, 4, 0, 1, 3).reshape(_S, _R, _D)


# speedup vs baseline: 1.0045x; 1.0045x over previous
import functools

import jax
import jax.numpy as jnp
from jax import lax
from jax.experimental import pallas as pl
from jax.experimental.pallas import tpu as pltpu
from jax.experimental.pallas import tpu_sc as plsc

_S = 4096
_R = 26
_D = 64
_NC, _NS = 2, 16
_NW = _NC * _NS
_SB = _S // _NW
_RC = 13
_CHUNK = _RC * _SB


def _embed_body(xt_hbm, table_hbm, out_hbm, idx_v, rows_a, rows_b, big_v,
                gsa, gsb, wsem_a, wsem_b, isem):
    w = lax.axis_index("s") * _NC + lax.axis_index("c")
    s0 = w * _SB
    lane = jax.lax.iota(jnp.int32, 16)
    dstc = [16 * m + lane for m in range(8)]

    ihs = [
        pltpu.async_copy(
            xt_hbm.at[r, pl.ds(s0, _SB)],
            idx_v.at[pl.ds(_SB * r, _SB)],
            isem,
        )
        for r in range(_R)
    ]
    for h in ihs:
        h.wait()

    subs = [(0, 7), (7, 6), (13, 7), (20, 6)]
    bufs = [(rows_a, gsa), (rows_b, gsb)]

    def gather(si):
        roff, nj = subs[si]
        buf, sem = bufs[si % 2]
        return pltpu.async_copy(
            table_hbm.at[idx_v.at[pl.ds(_SB * roff, _SB * nj)]],
            buf.at[pl.ds(0, _SB * nj), :],
            sem,
        )

    def transpose_sub(si):
        roff, nj = subs[si]
        rows_v = bufs[si % 2][0]

        def tile_loop(j, carry):
            r = roff + j
            rb = [j * _SB + 16 * m + lane for m in range(8)]
            for half in range(2):
                wsem = wsem_a if half == 0 else wsem_b

                @pl.when(j > 0)
                def _(half=half, wsem=wsem):
                    for ti in range(4 * half, 4 * half + 4):
                        pltpu.make_async_copy(
                            out_hbm.at[0],
                            big_v.at[pl.ds(8 * ti, 8), :],
                            wsem,
                        ).wait()

                for q in (2 * half, 2 * half + 1):
                    def diag_loop(i, cc, q=q, rb=rb):
                        for ii in range(2):
                            dg = q * 16 + ((2 * i + ii + lane) & 15)
                            for m0 in range(0, 8, 4):
                                vs = [
                                    plsc.load_gather(
                                        rows_v, [rb[m0 + t], dg]
                                    )
                                    for t in range(4)
                                ]
                                for t in range(4):
                                    plsc.store_scatter(
                                        big_v, [dg, dstc[m0 + t]], vs[t]
                                    )
                        return cc
                    lax.fori_loop(0, 8, diag_loop, 0)
                for ti in range(4 * half, 4 * half + 4):
                    pltpu.async_copy(
                        big_v.at[pl.ds(8 * ti, 8), :],
                        out_hbm.at[r * 256 + ti * 32 + w],
                        wsem,
                    )
            return carry

        lax.fori_loop(0, nj, tile_loop, 0)
        for ti in range(8):
            pltpu.make_async_copy(
                out_hbm.at[0],
                big_v.at[pl.ds(8 * ti, 8), :],
                wsem_a if ti < 4 else wsem_b,
            ).wait()

    hg = {0: gather(0)}
    for si in range(4):
        if si < 3:
            hg[si + 1] = gather(si + 1)
        hg[si].wait()
        transpose_sub(si)


def _pack_body(wt_hbm, tail_hbm, out2_hbm, in_a, in_b, out_a, out_b,
               inx_v, outx_v, tail_v, si_a, si_b, so_a, so_b):
    w = lax.axis_index("s") * _NC + lax.axis_index("c")
    lane = jax.lax.iota(jnp.int32, 16)
    rowc = [16 * m + lane for m in range(4)]
    scol = [[64 * a + 16 * q + lane for q in range(4)] for a in (0, 1)]

    def transpose_pair(i2, cc, in_v=None, out_v=None):
        for ii in range(2):
            i3 = 2 * i2 + ii
            usub = i3 >> 4
            dgs = (i3 + lane) & 15
            dg2 = 2 * dgs
            for p0 in (0, 16, 32, 48):
                rowv2 = dgs + (64 * usub + p0)
                for a in (0, 1):
                    col = dg2 + (128 * usub + 2 * p0 + a)
                    vs = [
                        plsc.load_gather(in_v, [rowc[q], col])
                        for q in range(4)
                    ]
                    for q in range(4):
                        plsc.store_scatter(
                            out_v, [rowv2, scol[a][q]], vs[q]
                        )
        return cc

    h_in, h_out = {}, {}
    bufs = [(in_a, out_a, si_a, so_a), (in_b, out_b, si_b, so_b)]
    c_base = w * 24 * 128
    h_in[0] = pltpu.async_copy(
        wt_hbm.at[:, pl.ds(c_base, 256)], in_a, si_a
    )
    for mm in range(12):
        in_v, out_v, si, so = bufs[mm % 2]
        if mm < 11:
            n_in = bufs[(mm + 1) % 2][0]
            h_in[mm + 1] = pltpu.async_copy(
                wt_hbm.at[:, pl.ds(c_base + (mm + 1) * 256, 256)],
                n_in,
                bufs[(mm + 1) % 2][2],
            )
        h_in[mm].wait()
        if mm >= 2:
            h_out[mm - 2].wait()
        lax.fori_loop(
            0, 16,
            functools.partial(transpose_pair, in_v=in_v, out_v=out_v),
            0,
        )
        h_out[mm] = pltpu.async_copy(
            out_v,
            out2_hbm.at[pl.ds((w * 24 + 2 * mm) * 64, 128), :],
            so,
        )
    h_out[10].wait()
    h_out[11].wait()

    @pl.when(w < 13)
    def _():
        pltpu.async_copy(
            wt_hbm.at[:, pl.ds((768 + w) * 128, 128)], inx_v, si_a
        ).wait()
        lax.fori_loop(
            0, 8,
            functools.partial(transpose_pair, in_v=inx_v, out_v=outx_v),
            0,
        )
        pltpu.async_copy(
            outx_v, out2_hbm.at[pl.ds((768 + w) * 64, 64), :], so_a
        ).wait()

    @pl.when(w == 31)
    def _():
        pltpu.async_copy(tail_hbm, tail_v, si_b).wait()
        pltpu.async_copy(
            tail_v, out2_hbm.at[pl.ds(49984, 16), :], so_b
        ).wait()


def _pack_table(weight):
    wt = weight.T
    tail = lax.slice(weight, (99968, 0), (100000, 64)).reshape(16, 128)
    mesh = plsc.VectorSubcoreMesh(core_axis_name="c", subcore_axis_name="s")
    k = pl.kernel(
        _pack_body,
        mesh=mesh,
        out_type=jax.ShapeDtypeStruct((50000, 128), jnp.float32),
        scratch_types=[
            pltpu.VMEM((_D, 256), jnp.float32),
            pltpu.VMEM((_D, 256), jnp.float32),
            pltpu.VMEM((128, 128), jnp.float32),
            pltpu.VMEM((128, 128), jnp.float32),
            pltpu.VMEM((_D, 128), jnp.float32),
            pltpu.VMEM((_D, 128), jnp.float32),
            pltpu.VMEM((16, 128), jnp.float32),
            pltpu.SemaphoreType.DMA,
            pltpu.SemaphoreType.DMA,
            pltpu.SemaphoreType.DMA,
            pltpu.SemaphoreType.DMA,
        ],
        compiler_params=pltpu.CompilerParams(
            use_tc_tiling_on_sc=True, needs_layout_passes=False
        ),
    )
    return k(wt, tail).reshape(100000, _D)


def kernel(x, weight):
    xt = x.T
    weight = _pack_table(weight)
    mesh = plsc.VectorSubcoreMesh(core_axis_name="c", subcore_axis_name="s")
    k = pl.kernel(
        _embed_body,
        mesh=mesh,
        out_type=jax.ShapeDtypeStruct((_R * 8 * _NW, 8, 128), jnp.float32),
        scratch_types=[
            pltpu.VMEM((_R * _SB,), jnp.int32),
            pltpu.VMEM((7 * _SB, _D), jnp.float32),
            pltpu.VMEM((7 * _SB, _D), jnp.float32),
            pltpu.VMEM((_D, 128), jnp.float32),
            pltpu.SemaphoreType.DMA,
            pltpu.SemaphoreType.DMA,
            pltpu.SemaphoreType.DMA,
            pltpu.SemaphoreType.DMA,
            pltpu.SemaphoreType.DMA,
        ],
        compiler_params=pltpu.CompilerParams(
            use_tc_tiling_on_sc=False, needs_layout_passes=False
        ),
    )
    out3 = k(xt, weight)
    t = out3.reshape(_R, 8, _NW, 8, 128)
    return t.transpose(2, 4, 0, 1, 3).reshape(_S, _R, _D)


# speedup vs baseline: 1.1401x; 1.1350x over previous
import functools

import jax
import jax.numpy as jnp
from jax import lax
from jax.experimental import pallas as pl
from jax.experimental.pallas import tpu as pltpu
from jax.experimental.pallas import tpu_sc as plsc

_S = 4096
_R = 26
_D = 64
_NC, _NS = 2, 16
_NW = _NC * _NS
_SB = _S // _NW
_RC = 13
_CHUNK = _RC * _SB


def _embed_body(xt_hbm, table_hbm, out_hbm, idx_v, rows_a, rows_b, big_v,
                gsa, gsb, wsem_a, wsem_b, isem):
    w = lax.axis_index("s") * _NC + lax.axis_index("c")
    s0 = w * _SB
    lane = jax.lax.iota(jnp.int32, 16)
    dstc = [16 * m + lane for m in range(8)]

    ihs = [
        pltpu.async_copy(
            xt_hbm.at[r, pl.ds(s0, _SB)],
            idx_v.at[pl.ds(_SB * r, _SB)],
            isem,
        )
        for r in range(_R)
    ]
    for h in ihs:
        h.wait()

    subs = [(0, 7), (7, 6), (13, 7), (20, 6)]
    bufs = [(rows_a, gsa), (rows_b, gsb)]

    def gather(si):
        roff, nj = subs[si]
        buf, sem = bufs[si % 2]
        return pltpu.async_copy(
            table_hbm.at[idx_v.at[pl.ds(_SB * roff, _SB * nj)]],
            buf.at[pl.ds(0, _SB * nj), :],
            sem,
        )

    def transpose_sub(si):
        roff, nj = subs[si]
        rows_v = bufs[si % 2][0]

        def tile_loop(j, carry):
            r = roff + j
            rb = [j * _SB + 16 * m + lane for m in range(8)]
            for half in range(2):
                wsem = wsem_a if half == 0 else wsem_b

                @pl.when(j > 0)
                def _(half=half, wsem=wsem):
                    for ti in range(4 * half, 4 * half + 4):
                        pltpu.make_async_copy(
                            out_hbm.at[0],
                            big_v.at[pl.ds(8 * ti, 8), :],
                            wsem,
                        ).wait()

                for q in (2 * half, 2 * half + 1):
                    def diag_loop(i, cc, q=q, rb=rb):
                        for ii in range(2):
                            dg = q * 16 + ((2 * i + ii + lane) & 15)
                            vs = [
                                plsc.load_gather(rows_v, [rb[m], dg])
                                for m in range(8)
                            ]
                            for m in range(8):
                                plsc.store_scatter(
                                    big_v, [dg, dstc[m]], vs[m]
                                )
                        return cc
                    lax.fori_loop(0, 8, diag_loop, 0)
                for ti in range(4 * half, 4 * half + 4):
                    pltpu.async_copy(
                        big_v.at[pl.ds(8 * ti, 8), :],
                        out_hbm.at[r * 256 + ti * 32 + w],
                        wsem,
                    )
            return carry

        lax.fori_loop(0, nj, tile_loop, 0)
        for ti in range(8):
            pltpu.make_async_copy(
                out_hbm.at[0],
                big_v.at[pl.ds(8 * ti, 8), :],
                wsem_a if ti < 4 else wsem_b,
            ).wait()

    hg = {0: gather(0)}
    for si in range(4):
        if si < 3:
            hg[si + 1] = gather(si + 1)
        hg[si].wait()
        transpose_sub(si)


def _pack_body(wt_hbm, tail_hbm, out2_hbm, in_a, in_b, out_a, out_b,
               inx_v, outx_v, tail_v, si_a, si_b, so_a, so_b):
    w = lax.axis_index("s") * _NC + lax.axis_index("c")
    lane = jax.lax.iota(jnp.int32, 16)
    rowc = [16 * m + lane for m in range(4)]
    scol = [[64 * a + 16 * q + lane for q in range(4)] for a in (0, 1)]

    def transpose_pair(i2, cc, in_v=None, out_v=None):
        for ii in range(2):
            i3 = 2 * i2 + ii
            usub = i3 >> 4
            dgs = (i3 + lane) & 15
            dg2 = 2 * dgs
            for p0 in (0, 16, 32, 48):
                rowv2 = dgs + (64 * usub + p0)
                cols = [
                    dg2 + (128 * usub + 2 * p0 + a) for a in (0, 1)
                ]
                vs = [
                    plsc.load_gather(in_v, [rowc[q], cols[a]])
                    for a in (0, 1)
                    for q in range(4)
                ]
                for a in (0, 1):
                    for q in range(4):
                        plsc.store_scatter(
                            out_v, [rowv2, scol[a][q]], vs[4 * a + q]
                        )
        return cc

    h_in, h_out = {}, {}
    bufs = [(in_a, out_a, si_a, so_a), (in_b, out_b, si_b, so_b)]
    c_base = w * 24 * 128
    h_in[0] = pltpu.async_copy(
        wt_hbm.at[:, pl.ds(c_base, 256)], in_a, si_a
    )
    for mm in range(12):
        in_v, out_v, si, so = bufs[mm % 2]
        if mm < 11:
            n_in = bufs[(mm + 1) % 2][0]
            h_in[mm + 1] = pltpu.async_copy(
                wt_hbm.at[:, pl.ds(c_base + (mm + 1) * 256, 256)],
                n_in,
                bufs[(mm + 1) % 2][2],
            )
        h_in[mm].wait()
        if mm >= 2:
            h_out[mm - 2].wait()
        lax.fori_loop(
            0, 16,
            functools.partial(transpose_pair, in_v=in_v, out_v=out_v),
            0,
        )
        h_out[mm] = pltpu.async_copy(
            out_v,
            out2_hbm.at[pl.ds((w * 24 + 2 * mm) * 64, 128), :],
            so,
        )
    h_out[10].wait()
    h_out[11].wait()

    @pl.when(w < 13)
    def _():
        pltpu.async_copy(
            wt_hbm.at[:, pl.ds((768 + w) * 128, 128)], inx_v, si_a
        ).wait()
        lax.fori_loop(
            0, 8,
            functools.partial(transpose_pair, in_v=inx_v, out_v=outx_v),
            0,
        )
        pltpu.async_copy(
            outx_v, out2_hbm.at[pl.ds((768 + w) * 64, 64), :], so_a
        ).wait()

    @pl.when(w == 31)
    def _():
        pltpu.async_copy(tail_hbm, tail_v, si_b).wait()
        pltpu.async_copy(
            tail_v, out2_hbm.at[pl.ds(49984, 16), :], so_b
        ).wait()


def _pack_table(weight):
    wt = weight.T
    tail = lax.slice(weight, (99968, 0), (100000, 64)).reshape(16, 128)
    mesh = plsc.VectorSubcoreMesh(core_axis_name="c", subcore_axis_name="s")
    k = pl.kernel(
        _pack_body,
        mesh=mesh,
        out_type=jax.ShapeDtypeStruct((50000, 128), jnp.float32),
        scratch_types=[
            pltpu.VMEM((_D, 256), jnp.float32),
            pltpu.VMEM((_D, 256), jnp.float32),
            pltpu.VMEM((128, 128), jnp.float32),
            pltpu.VMEM((128, 128), jnp.float32),
            pltpu.VMEM((_D, 128), jnp.float32),
            pltpu.VMEM((_D, 128), jnp.float32),
            pltpu.VMEM((16, 128), jnp.float32),
            pltpu.SemaphoreType.DMA,
            pltpu.SemaphoreType.DMA,
            pltpu.SemaphoreType.DMA,
            pltpu.SemaphoreType.DMA,
        ],
        compiler_params=pltpu.CompilerParams(
            use_tc_tiling_on_sc=True, needs_layout_passes=False
        ),
    )
    return k(wt, tail).reshape(100000, _D)


def kernel(x, weight):
    xt = x.T
    weight = _pack_table(weight)
    mesh = plsc.VectorSubcoreMesh(core_axis_name="c", subcore_axis_name="s")
    k = pl.kernel(
        _embed_body,
        mesh=mesh,
        out_type=jax.ShapeDtypeStruct((_R * 8 * _NW, 8, 128), jnp.float32),
        scratch_types=[
            pltpu.VMEM((_R * _SB,), jnp.int32),
            pltpu.VMEM((7 * _SB, _D), jnp.float32),
            pltpu.VMEM((7 * _SB, _D), jnp.float32),
            pltpu.VMEM((_D, 128), jnp.float32),
            pltpu.SemaphoreType.DMA,
            pltpu.SemaphoreType.DMA,
            pltpu.SemaphoreType.DMA,
            pltpu.SemaphoreType.DMA,
            pltpu.SemaphoreType.DMA,
        ],
        compiler_params=pltpu.CompilerParams(
            use_tc_tiling_on_sc=False, needs_layout_passes=False
        ),
    )
    out3 = k(xt, weight)
    t = out3.reshape(_R, 8, _NW, 8, 128)
    return t.transpose(2, 4, 0, 1, 3).reshape(_S, _R, _D)


# speedup vs baseline: 1.1730x; 1.0289x over previous
import functools

import jax
import jax.numpy as jnp
from jax import lax
from jax.experimental import pallas as pl
from jax.experimental.pallas import tpu as pltpu
from jax.experimental.pallas import tpu_sc as plsc

_S = 4096
_R = 26
_D = 64
_NC, _NS = 2, 16
_NW = _NC * _NS
_SB = _S // _NW
_RC = 13
_CHUNK = _RC * _SB


def _embed_body(xt_hbm, table_hbm, out_hbm, idx_v, rows_a, rows_b, big_v,
                gsa, gsb, wsem_a, wsem_b, isem):
    w = lax.axis_index("s") * _NC + lax.axis_index("c")
    s0 = w * _SB
    lane = jax.lax.iota(jnp.int32, 16)
    dstc = [16 * m + lane for m in range(8)]

    ihs = [
        pltpu.async_copy(
            xt_hbm.at[r, pl.ds(s0, _SB)],
            idx_v.at[pl.ds(_SB * r, _SB)],
            isem,
        )
        for r in range(_R)
    ]
    for h in ihs:
        h.wait()

    subs = [(0, 7), (7, 6), (13, 7), (20, 6)]
    bufs = [(rows_a, gsa), (rows_b, gsb)]

    def gather(si):
        roff, nj = subs[si]
        buf, sem = bufs[si % 2]
        return pltpu.async_copy(
            table_hbm.at[idx_v.at[pl.ds(_SB * roff, _SB * nj)]],
            buf.at[pl.ds(0, _SB * nj), :],
            sem,
        )

    def transpose_sub(si):
        roff, nj = subs[si]
        rows_v = bufs[si % 2][0]

        def tile_loop(j, carry):
            r = roff + j
            rb = [j * _SB + 16 * m + lane for m in range(8)]
            for half in range(2):
                wsem = wsem_a if half == 0 else wsem_b

                @pl.when(j > 0)
                def _(half=half, wsem=wsem):
                    for ti in range(4 * half, 4 * half + 4):
                        pltpu.make_async_copy(
                            out_hbm.at[0],
                            big_v.at[pl.ds(8 * ti, 8), :],
                            wsem,
                        ).wait()

                for q in (2 * half, 2 * half + 1):
                    def diag_loop(i, cc, q=q, rb=rb):
                        for ii in range(2):
                            dg = q * 16 + ((2 * i + ii + lane) & 15)
                            vs = [
                                plsc.load_gather(rows_v, [rb[m], dg])
                                for m in range(8)
                            ]
                            for m in range(8):
                                plsc.store_scatter(
                                    big_v, [dg, dstc[m]], vs[m]
                                )
                        return cc
                    lax.fori_loop(0, 8, diag_loop, 0)
                for ti in range(4 * half, 4 * half + 4):
                    pltpu.async_copy(
                        big_v.at[pl.ds(8 * ti, 8), :],
                        out_hbm.at[r * 256 + ti * 32 + w],
                        wsem,
                    )
            return carry

        lax.fori_loop(0, nj, tile_loop, 0)
        for ti in range(8):
            pltpu.make_async_copy(
                out_hbm.at[0],
                big_v.at[pl.ds(8 * ti, 8), :],
                wsem_a if ti < 4 else wsem_b,
            ).wait()

    hg = {0: gather(0)}
    for si in range(4):
        if si < 3:
            hg[si + 1] = gather(si + 1)
        hg[si].wait()
        transpose_sub(si)


def _pack_body(wt_hbm, tail_hbm, out2_hbm, in_a, in_b, out_a, out_b,
               inx_v, outx_v, tail_v, si_a, si_b, so_a, so_b):
    w = lax.axis_index("s") * _NC + lax.axis_index("c")
    lane = jax.lax.iota(jnp.int32, 16)
    rowc = [16 * m + lane for m in range(4)]
    scol = [[64 * a + 16 * q + lane for q in range(4)] for a in (0, 1)]

    def transpose_pair(i2, cc, in_v=None, out_v=None):
        for ii in range(2):
            i3 = 2 * i2 + ii
            usub = i3 >> 4
            dgs = (i3 + lane) & 15
            dg2 = 2 * dgs
            for p0 in (0, 16, 32, 48):
                rowv2 = dgs + (64 * usub + p0)
                cols = [
                    dg2 + (128 * usub + 2 * p0 + a) for a in (0, 1)
                ]
                vs = [
                    plsc.load_gather(in_v, [rowc[q], cols[a]])
                    for a in (0, 1)
                    for q in range(4)
                ]
                for a in (0, 1):
                    for q in range(4):
                        plsc.store_scatter(
                            out_v, [rowv2, scol[a][q]], vs[4 * a + q]
                        )
        return cc

    h_in, h_out = {}, {}
    bufs = [(in_a, out_a, si_a, so_a), (in_b, out_b, si_b, so_b)]
    c_base = w * 24 * 128
    h_in[0] = pltpu.async_copy(
        wt_hbm.at[:, pl.ds(c_base, 384)], in_a, si_a
    )
    for mm in range(8):
        in_v, out_v, si, so = bufs[mm % 2]
        if mm < 7:
            n_in = bufs[(mm + 1) % 2][0]
            h_in[mm + 1] = pltpu.async_copy(
                wt_hbm.at[:, pl.ds(c_base + (mm + 1) * 384, 384)],
                n_in,
                bufs[(mm + 1) % 2][2],
            )
        h_in[mm].wait()
        if mm >= 2:
            h_out[mm - 2].wait()
        lax.fori_loop(
            0, 24,
            functools.partial(transpose_pair, in_v=in_v, out_v=out_v),
            0,
        )
        h_out[mm] = pltpu.async_copy(
            out_v,
            out2_hbm.at[pl.ds((w * 24 + 3 * mm) * 64, 192), :],
            so,
        )
    h_out[6].wait()
    h_out[7].wait()

    @pl.when(w < 13)
    def _():
        pltpu.async_copy(
            wt_hbm.at[:, pl.ds((768 + w) * 128, 128)], inx_v, si_a
        ).wait()
        lax.fori_loop(
            0, 8,
            functools.partial(transpose_pair, in_v=inx_v, out_v=outx_v),
            0,
        )
        pltpu.async_copy(
            outx_v, out2_hbm.at[pl.ds((768 + w) * 64, 64), :], so_a
        ).wait()

    @pl.when(w == 31)
    def _():
        pltpu.async_copy(tail_hbm, tail_v, si_b).wait()
        pltpu.async_copy(
            tail_v, out2_hbm.at[pl.ds(49984, 16), :], so_b
        ).wait()


def _pack_table(weight):
    wt = weight.T
    tail = lax.slice(weight, (99968, 0), (100000, 64)).reshape(16, 128)
    mesh = plsc.VectorSubcoreMesh(core_axis_name="c", subcore_axis_name="s")
    k = pl.kernel(
        _pack_body,
        mesh=mesh,
        out_type=jax.ShapeDtypeStruct((50000, 128), jnp.float32),
        scratch_types=[
            pltpu.VMEM((_D, 384), jnp.float32),
            pltpu.VMEM((_D, 384), jnp.float32),
            pltpu.VMEM((192, 128), jnp.float32),
            pltpu.VMEM((192, 128), jnp.float32),
            pltpu.VMEM((_D, 128), jnp.float32),
            pltpu.VMEM((_D, 128), jnp.float32),
            pltpu.VMEM((16, 128), jnp.float32),
            pltpu.SemaphoreType.DMA,
            pltpu.SemaphoreType.DMA,
            pltpu.SemaphoreType.DMA,
            pltpu.SemaphoreType.DMA,
        ],
        compiler_params=pltpu.CompilerParams(
            use_tc_tiling_on_sc=True, needs_layout_passes=False
        ),
    )
    return k(wt, tail).reshape(100000, _D)


def kernel(x, weight):
    xt = x.T
    weight = _pack_table(weight)
    mesh = plsc.VectorSubcoreMesh(core_axis_name="c", subcore_axis_name="s")
    k = pl.kernel(
        _embed_body,
        mesh=mesh,
        out_type=jax.ShapeDtypeStruct((_R * 8 * _NW, 8, 128), jnp.float32),
        scratch_types=[
            pltpu.VMEM((_R * _SB,), jnp.int32),
            pltpu.VMEM((7 * _SB, _D), jnp.float32),
            pltpu.VMEM((7 * _SB, _D), jnp.float32),
            pltpu.VMEM((_D, 128), jnp.float32),
            pltpu.SemaphoreType.DMA,
            pltpu.SemaphoreType.DMA,
            pltpu.SemaphoreType.DMA,
            pltpu.SemaphoreType.DMA,
            pltpu.SemaphoreType.DMA,
        ],
        compiler_params=pltpu.CompilerParams(
            use_tc_tiling_on_sc=False, needs_layout_passes=False
        ),
    )
    out3 = k(xt, weight)
    t = out3.reshape(_R, 8, _NW, 8, 128)
    return t.transpose(2, 4, 0, 1, 3).reshape(_S, _R, _D)


# speedup vs baseline: 1.1915x; 1.0158x over previous
import functools

import jax
import jax.numpy as jnp
from jax import lax
from jax.experimental import pallas as pl
from jax.experimental.pallas import tpu as pltpu
from jax.experimental.pallas import tpu_sc as plsc

_S = 4096
_R = 26
_D = 64
_NC, _NS = 2, 16
_NW = _NC * _NS
_SB = _S // _NW
_RC = 13
_CHUNK = _RC * _SB


def _embed_body(xt_hbm, table_hbm, out_hbm, idx_v, rows_a, rows_b, big_v,
                gsa, gsb, wsem_a, wsem_b, isem):
    w = lax.axis_index("s") * _NC + lax.axis_index("c")
    s0 = w * _SB
    lane = jax.lax.iota(jnp.int32, 16)
    dstc = [16 * m + lane for m in range(8)]

    ihs = [
        pltpu.async_copy(
            xt_hbm.at[r, pl.ds(s0, _SB)],
            idx_v.at[pl.ds(_SB * r, _SB)],
            isem,
        )
        for r in range(_R)
    ]
    for h in ihs:
        h.wait()

    subs = [(0, 7), (7, 6), (13, 7), (20, 6)]
    bufs = [(rows_a, gsa), (rows_b, gsb)]

    def gather(si):
        roff, nj = subs[si]
        buf, sem = bufs[si % 2]
        return pltpu.async_copy(
            table_hbm.at[idx_v.at[pl.ds(_SB * roff, _SB * nj)]],
            buf.at[pl.ds(0, _SB * nj), :],
            sem,
        )

    def transpose_sub(si):
        roff, nj = subs[si]
        rows_v = bufs[si % 2][0]

        def tile_loop(j, carry):
            r = roff + j
            rb = [j * _SB + 16 * m + lane for m in range(8)]
            for half in range(2):
                wsem = wsem_a if half == 0 else wsem_b

                @pl.when(j > 0)
                def _(half=half, wsem=wsem):
                    for ti in range(4 * half, 4 * half + 4):
                        pltpu.make_async_copy(
                            out_hbm.at[0],
                            big_v.at[pl.ds(8 * ti, 8), :],
                            wsem,
                        ).wait()

                for q in (2 * half, 2 * half + 1):
                    def diag_loop(i, cc, q=q, rb=rb):
                        dgs = [
                            q * 16 + ((2 * i + ii + lane) & 15)
                            for ii in range(2)
                        ]
                        vs = [
                            plsc.load_gather(rows_v, [rb[m], dgs[ii]])
                            for ii in range(2)
                            for m in range(8)
                        ]
                        for ii in range(2):
                            for m in range(8):
                                plsc.store_scatter(
                                    big_v, [dgs[ii], dstc[m]], vs[8 * ii + m]
                                )
                        return cc
                    lax.fori_loop(0, 8, diag_loop, 0)
                for ti in range(4 * half, 4 * half + 4):
                    pltpu.async_copy(
                        big_v.at[pl.ds(8 * ti, 8), :],
                        out_hbm.at[r * 256 + ti * 32 + w],
                        wsem,
                    )
            return carry

        lax.fori_loop(0, nj, tile_loop, 0)
        for ti in range(8):
            pltpu.make_async_copy(
                out_hbm.at[0],
                big_v.at[pl.ds(8 * ti, 8), :],
                wsem_a if ti < 4 else wsem_b,
            ).wait()

    hg = {0: gather(0)}
    for si in range(4):
        if si < 3:
            hg[si + 1] = gather(si + 1)
        hg[si].wait()
        transpose_sub(si)


def _pack_body(wt_hbm, tail_hbm, out2_hbm, in_a, in_b, out_a, out_b,
               inx_v, outx_v, tail_v, si_a, si_b, so_a, so_b):
    w = lax.axis_index("s") * _NC + lax.axis_index("c")
    lane = jax.lax.iota(jnp.int32, 16)
    rowc = [16 * m + lane for m in range(4)]
    scol = [[64 * a + 16 * q + lane for q in range(4)] for a in (0, 1)]

    def transpose_pair(i2, cc, in_v=None, out_v=None):
        for ii in range(2):
            i3 = 2 * i2 + ii
            usub = i3 >> 4
            dgs = (i3 + lane) & 15
            dg2 = 2 * dgs
            for p0 in (0, 16, 32, 48):
                rowv2 = dgs + (64 * usub + p0)
                cols = [
                    dg2 + (128 * usub + 2 * p0 + a) for a in (0, 1)
                ]
                vs = [
                    plsc.load_gather(in_v, [rowc[q], cols[a]])
                    for a in (0, 1)
                    for q in range(4)
                ]
                for a in (0, 1):
                    for q in range(4):
                        plsc.store_scatter(
                            out_v, [rowv2, scol[a][q]], vs[4 * a + q]
                        )
        return cc

    h_in, h_out = {}, {}
    bufs = [(in_a, out_a, si_a, so_a), (in_b, out_b, si_b, so_b)]
    c_base = w * 24 * 128
    h_in[0] = pltpu.async_copy(
        wt_hbm.at[:, pl.ds(c_base, 384)], in_a, si_a
    )
    for mm in range(8):
        in_v, out_v, si, so = bufs[mm % 2]
        if mm < 7:
            n_in = bufs[(mm + 1) % 2][0]
            h_in[mm + 1] = pltpu.async_copy(
                wt_hbm.at[:, pl.ds(c_base + (mm + 1) * 384, 384)],
                n_in,
                bufs[(mm + 1) % 2][2],
            )
        h_in[mm].wait()
        if mm >= 2:
            h_out[mm - 2].wait()
        lax.fori_loop(
            0, 24,
            functools.partial(transpose_pair, in_v=in_v, out_v=out_v),
            0,
        )
        h_out[mm] = pltpu.async_copy(
            out_v,
            out2_hbm.at[pl.ds((w * 24 + 3 * mm) * 64, 192), :],
            so,
        )
    h_out[6].wait()
    h_out[7].wait()

    @pl.when(w < 13)
    def _():
        pltpu.async_copy(
            wt_hbm.at[:, pl.ds((768 + w) * 128, 128)], inx_v, si_a
        ).wait()
        lax.fori_loop(
            0, 8,
            functools.partial(transpose_pair, in_v=inx_v, out_v=outx_v),
            0,
        )
        pltpu.async_copy(
            outx_v, out2_hbm.at[pl.ds((768 + w) * 64, 64), :], so_a
        ).wait()

    @pl.when(w == 31)
    def _():
        pltpu.async_copy(tail_hbm, tail_v, si_b).wait()
        pltpu.async_copy(
            tail_v, out2_hbm.at[pl.ds(49984, 16), :], so_b
        ).wait()


def _pack_table(weight):
    wt = weight.T
    tail = lax.slice(weight, (99968, 0), (100000, 64)).reshape(16, 128)
    mesh = plsc.VectorSubcoreMesh(core_axis_name="c", subcore_axis_name="s")
    k = pl.kernel(
        _pack_body,
        mesh=mesh,
        out_type=jax.ShapeDtypeStruct((50000, 128), jnp.float32),
        scratch_types=[
            pltpu.VMEM((_D, 384), jnp.float32),
            pltpu.VMEM((_D, 384), jnp.float32),
            pltpu.VMEM((192, 128), jnp.float32),
            pltpu.VMEM((192, 128), jnp.float32),
            pltpu.VMEM((_D, 128), jnp.float32),
            pltpu.VMEM((_D, 128), jnp.float32),
            pltpu.VMEM((16, 128), jnp.float32),
            pltpu.SemaphoreType.DMA,
            pltpu.SemaphoreType.DMA,
            pltpu.SemaphoreType.DMA,
            pltpu.SemaphoreType.DMA,
        ],
        compiler_params=pltpu.CompilerParams(
            use_tc_tiling_on_sc=True, needs_layout_passes=False
        ),
    )
    return k(wt, tail).reshape(100000, _D)


def kernel(x, weight):
    xt = x.T
    weight = _pack_table(weight)
    mesh = plsc.VectorSubcoreMesh(core_axis_name="c", subcore_axis_name="s")
    k = pl.kernel(
        _embed_body,
        mesh=mesh,
        out_type=jax.ShapeDtypeStruct((_R * 8 * _NW, 8, 128), jnp.float32),
        scratch_types=[
            pltpu.VMEM((_R * _SB,), jnp.int32),
            pltpu.VMEM((7 * _SB, _D), jnp.float32),
            pltpu.VMEM((7 * _SB, _D), jnp.float32),
            pltpu.VMEM((_D, 128), jnp.float32),
            pltpu.SemaphoreType.DMA,
            pltpu.SemaphoreType.DMA,
            pltpu.SemaphoreType.DMA,
            pltpu.SemaphoreType.DMA,
            pltpu.SemaphoreType.DMA,
        ],
        compiler_params=pltpu.CompilerParams(
            use_tc_tiling_on_sc=False, needs_layout_passes=False
        ),
    )
    out3 = k(xt, weight)
    t = out3.reshape(_R, 8, _NW, 8, 128)
    return t.transpose(2, 4, 0, 1, 3).reshape(_S, _R, _D)


# speedup vs baseline: 1.2091x; 1.0147x over previous
import functools

import jax
import jax.numpy as jnp
from jax import lax
from jax.experimental import pallas as pl
from jax.experimental.pallas import tpu as pltpu
from jax.experimental.pallas import tpu_sc as plsc

_S = 4096
_R = 26
_D = 64
_NC, _NS = 2, 16
_NW = _NC * _NS
_SB = _S // _NW
_RC = 13
_CHUNK = _RC * _SB


def _embed_body(xt_hbm, table_hbm, out_hbm, idx_v, rows_a, rows_b, big_v,
                gsa, gsb, wsem_a, wsem_b, isem):
    w = lax.axis_index("s") * _NC + lax.axis_index("c")
    s0 = w * _SB
    lane = jax.lax.iota(jnp.int32, 16)
    dstc = [16 * m + lane for m in range(8)]

    ihs = [
        pltpu.async_copy(
            xt_hbm.at[r, pl.ds(s0, _SB)],
            idx_v.at[pl.ds(_SB * r, _SB)],
            isem,
        )
        for r in range(_R)
    ]
    for h in ihs:
        h.wait()

    subs = [(0, 7), (7, 6), (13, 7), (20, 6)]
    bufs = [(rows_a, gsa), (rows_b, gsb)]

    def gather(si):
        roff, nj = subs[si]
        buf, sem = bufs[si % 2]
        return pltpu.async_copy(
            table_hbm.at[idx_v.at[pl.ds(_SB * roff, _SB * nj)]],
            buf.at[pl.ds(0, _SB * nj), :],
            sem,
        )

    def transpose_sub(si):
        roff, nj = subs[si]
        rows_v = bufs[si % 2][0]

        def tile_loop(j, carry):
            r = roff + j
            rb = [j * _SB + 16 * m + lane for m in range(8)]
            for half in range(2):
                wsem = wsem_a if half == 0 else wsem_b

                @pl.when(j > 0)
                def _(half=half, wsem=wsem):
                    for ti in range(4 * half, 4 * half + 4):
                        pltpu.make_async_copy(
                            out_hbm.at[0],
                            big_v.at[pl.ds(8 * ti, 8), :],
                            wsem,
                        ).wait()

                for q in (2 * half, 2 * half + 1):
                    def diag_loop(i, cc, q=q, rb=rb):
                        dgs = [
                            q * 16 + ((2 * i + ii + lane) & 15)
                            for ii in range(2)
                        ]
                        vs = [
                            plsc.load_gather(rows_v, [rb[m], dgs[ii]])
                            for ii in range(2)
                            for m in range(8)
                        ]
                        for ii in range(2):
                            for m in range(8):
                                plsc.store_scatter(
                                    big_v, [dgs[ii], dstc[m]], vs[8 * ii + m]
                                )
                        return cc
                    lax.fori_loop(0, 8, diag_loop, 0)
                for ti in range(4 * half, 4 * half + 4):
                    pltpu.async_copy(
                        big_v.at[pl.ds(8 * ti, 8), :],
                        out_hbm.at[r * 256 + ti * 32 + w],
                        wsem,
                    )
            return carry

        lax.fori_loop(0, nj, tile_loop, 0)
        for ti in range(8):
            pltpu.make_async_copy(
                out_hbm.at[0],
                big_v.at[pl.ds(8 * ti, 8), :],
                wsem_a if ti < 4 else wsem_b,
            ).wait()

    hg = {0: gather(0)}
    for si in range(4):
        if si < 3:
            hg[si + 1] = gather(si + 1)
        hg[si].wait()
        transpose_sub(si)


def _pack_body(wt_hbm, tail_hbm, out2_hbm, in_a, in_b, out_a, out_b,
               inx_v, outx_v, tail_v, si_a, si_b, so_a, so_b):
    w = lax.axis_index("s") * _NC + lax.axis_index("c")
    lane = jax.lax.iota(jnp.int32, 16)
    rowc = [16 * m + lane for m in range(4)]
    scol = [[64 * a + 16 * q + lane for q in range(4)] for a in (0, 1)]

    def transpose_pair(i2, cc, in_v=None, out_v=None):
        for ii in range(2):
            i3 = 2 * i2 + ii
            usub = i3 >> 4
            dgs = (i3 + lane) & 15
            dg2 = 2 * dgs
            for pp in range(2):
                p0s = (0, 16) if pp == 0 else (32, 48)
                rowv2 = [dgs + (64 * usub + p0) for p0 in p0s]
                cols = [
                    dg2 + (128 * usub + 2 * p0 + a)
                    for p0 in p0s
                    for a in (0, 1)
                ]
                vs = [
                    plsc.load_gather(in_v, [rowc[q], cols[2 * pi + a]])
                    for pi in range(2)
                    for a in (0, 1)
                    for q in range(4)
                ]
                for pi in range(2):
                    for a in (0, 1):
                        for q in range(4):
                            plsc.store_scatter(
                                out_v,
                                [rowv2[pi], scol[a][q]],
                                vs[8 * pi + 4 * a + q],
                            )
        return cc

    h_in, h_out = {}, {}
    bufs = [(in_a, out_a, si_a, so_a), (in_b, out_b, si_b, so_b)]
    c_base = w * 24 * 128
    h_in[0] = pltpu.async_copy(
        wt_hbm.at[:, pl.ds(c_base, 384)], in_a, si_a
    )
    for mm in range(8):
        in_v, out_v, si, so = bufs[mm % 2]
        if mm < 7:
            n_in = bufs[(mm + 1) % 2][0]
            h_in[mm + 1] = pltpu.async_copy(
                wt_hbm.at[:, pl.ds(c_base + (mm + 1) * 384, 384)],
                n_in,
                bufs[(mm + 1) % 2][2],
            )
        h_in[mm].wait()
        if mm >= 2:
            h_out[mm - 2].wait()
        lax.fori_loop(
            0, 24,
            functools.partial(transpose_pair, in_v=in_v, out_v=out_v),
            0,
        )
        h_out[mm] = pltpu.async_copy(
            out_v,
            out2_hbm.at[pl.ds((w * 24 + 3 * mm) * 64, 192), :],
            so,
        )
    h_out[6].wait()
    h_out[7].wait()

    @pl.when(w < 13)
    def _():
        pltpu.async_copy(
            wt_hbm.at[:, pl.ds((768 + w) * 128, 128)], inx_v, si_a
        ).wait()
        lax.fori_loop(
            0, 8,
            functools.partial(transpose_pair, in_v=inx_v, out_v=outx_v),
            0,
        )
        pltpu.async_copy(
            outx_v, out2_hbm.at[pl.ds((768 + w) * 64, 64), :], so_a
        ).wait()

    @pl.when(w == 31)
    def _():
        pltpu.async_copy(tail_hbm, tail_v, si_b).wait()
        pltpu.async_copy(
            tail_v, out2_hbm.at[pl.ds(49984, 16), :], so_b
        ).wait()


def _pack_table(weight):
    wt = weight.T
    tail = lax.slice(weight, (99968, 0), (100000, 64)).reshape(16, 128)
    mesh = plsc.VectorSubcoreMesh(core_axis_name="c", subcore_axis_name="s")
    k = pl.kernel(
        _pack_body,
        mesh=mesh,
        out_type=jax.ShapeDtypeStruct((50000, 128), jnp.float32),
        scratch_types=[
            pltpu.VMEM((_D, 384), jnp.float32),
            pltpu.VMEM((_D, 384), jnp.float32),
            pltpu.VMEM((192, 128), jnp.float32),
            pltpu.VMEM((192, 128), jnp.float32),
            pltpu.VMEM((_D, 128), jnp.float32),
            pltpu.VMEM((_D, 128), jnp.float32),
            pltpu.VMEM((16, 128), jnp.float32),
            pltpu.SemaphoreType.DMA,
            pltpu.SemaphoreType.DMA,
            pltpu.SemaphoreType.DMA,
            pltpu.SemaphoreType.DMA,
        ],
        compiler_params=pltpu.CompilerParams(
            use_tc_tiling_on_sc=True, needs_layout_passes=False
        ),
    )
    return k(wt, tail).reshape(100000, _D)


def kernel(x, weight):
    xt = x.T
    weight = _pack_table(weight)
    mesh = plsc.VectorSubcoreMesh(core_axis_name="c", subcore_axis_name="s")
    k = pl.kernel(
        _embed_body,
        mesh=mesh,
        out_type=jax.ShapeDtypeStruct((_R * 8 * _NW, 8, 128), jnp.float32),
        scratch_types=[
            pltpu.VMEM((_R * _SB,), jnp.int32),
            pltpu.VMEM((7 * _SB, _D), jnp.float32),
            pltpu.VMEM((7 * _SB, _D), jnp.float32),
            pltpu.VMEM((_D, 128), jnp.float32),
            pltpu.SemaphoreType.DMA,
            pltpu.SemaphoreType.DMA,
            pltpu.SemaphoreType.DMA,
            pltpu.SemaphoreType.DMA,
            pltpu.SemaphoreType.DMA,
        ],
        compiler_params=pltpu.CompilerParams(
            use_tc_tiling_on_sc=False, needs_layout_passes=False
        ),
    )
    out3 = k(xt, weight)
    t = out3.reshape(_R, 8, _NW, 8, 128)
    return t.transpose(2, 4, 0, 1, 3).reshape(_S, _R, _D)
